# Initial kernel scaffold; baseline (speedup 1.0000x reference)
import functools

import jax
import jax.numpy as jnp
from jax import lax
from jax.experimental import pallas as pl
from jax.experimental.pallas import tpu as pltpu
from jax.experimental.pallas import tpu_sc as plsc

_B = 1024
_R = 50
_D = 10
_DP = 16
_NPAD = 1024
_LANES = 16


def _tc_body(x_ref, w_ref, b_ref, cpm_ref, huff_ref, y_ref, enc_ref):
    z = jnp.dot(x_ref[...], w_ref[...], preferred_element_type=jnp.float32)
    y_ref[...] = jax.nn.sigmoid(z + b_ref[...])
    enc_ref[...] = cpm_ref[...] + _NPAD * huff_ref[...]


def _tc_stage(x, w_t, b_row, cpm_pad, huff_pad):
    return pl.pallas_call(
        _tc_body,
        out_shape=(
            jax.ShapeDtypeStruct((_B, _NPAD), jnp.float32),
            jax.ShapeDtypeStruct((_NPAD, _DP), jnp.int32),
        ),
    )(x, w_t, b_row, cpm_pad, huff_pad)


def _make_sc_gather(n_workers):
    rows_per_w = _B // n_workers
    n_chunks = (rows_per_w * _R) // _LANES
    mesh = plsc.VectorSubcoreMesh(core_axis_name="c", subcore_axis_name="s")

    @functools.partial(
        pl.kernel,
        mesh=mesh,
        out_type=jax.ShapeDtypeStruct((_B, _R), jnp.float32),
        scratch_types=[
            pltpu.VMEM((rows_per_w, _NPAD), jnp.float32),
            pltpu.VMEM((_NPAD, _DP), jnp.int32),
            pltpu.VMEM((rows_per_w, _R), jnp.int32),
            pltpu.VMEM((rows_per_w, _R), jnp.float32),
        ],
    )
    def sc_gather(y_hbm, enc_hbm, tc_hbm, out_hbm, y_v, enc_v, tc_v, out_v):
        wid = lax.axis_index("s") * 2 + lax.axis_index("c")
        base = wid * rows_per_w
        pltpu.sync_copy(y_hbm.at[pl.ds(base, rows_per_w)], y_v)
        pltpu.sync_copy(enc_hbm, enc_v)
        pltpu.sync_copy(tc_hbm.at[pl.ds(base, rows_per_w)], tc_v)

        def chunk(j, carry):
            pos = j * _LANES + lax.iota(jnp.int32, _LANES)
            b16 = pos // _R
            r16 = pos - b16 * _R
            tc16 = plsc.load_gather(tc_v, [b16, r16])
            prod = jnp.ones((_LANES,), jnp.float32)
            for t in range(_D):
                tcol = jnp.full((_LANES,), t, jnp.int32)
                e = plsc.load_gather(enc_v, [tc16, tcol])
                node = jnp.bitwise_and(e, _NPAD - 1)
                c = (e >> 10).astype(jnp.float32)
                yv = plsc.load_gather(y_v, [b16, node])
                prod = prod * (c + yv - 2.0 * c * yv)
            plsc.store_scatter(out_v, [b16, r16], prod)
            return carry

        lax.fori_loop(0, n_chunks, chunk, 0)
        pltpu.sync_copy(out_v, out_hbm.at[pl.ds(base, rows_per_w)])

    return sc_gather


def kernel(input_vector, target_classes, W, b, class_path_map, huffman_codes):
    n_nodes = W.shape[0]
    w_t = jnp.zeros((input_vector.shape[1], _NPAD), jnp.float32)
    w_t = w_t.at[:, :n_nodes].set(W[:, :, 0].T)
    b_row = jnp.zeros((1, _NPAD), jnp.float32).at[0, :n_nodes].set(b[:, 0])
    cpm_pad = jnp.zeros((_NPAD, _DP), jnp.int32)
    cpm_pad = cpm_pad.at[: class_path_map.shape[0], :_D].set(class_path_map)
    huff_pad = jnp.zeros((_NPAD, _DP), jnp.int32)
    huff_pad = huff_pad.at[: huffman_codes.shape[0], :_D].set(huffman_codes)

    y_all, enc = _tc_stage(input_vector, w_t, b_row, cpm_pad, huff_pad)

    info = plsc.get_sparse_core_info()
    n_workers = info.num_cores * info.num_subcores
    out = _make_sc_gather(n_workers)(y_all, enc, target_classes.astype(jnp.int32))
    return out


# speedup vs baseline: 108.5138x; 108.5138x over previous
import functools

import jax
import jax.numpy as jnp
from jax import lax
from jax.experimental import pallas as pl
from jax.experimental.pallas import tpu as pltpu
from jax.experimental.pallas import tpu_sc as plsc

_B = 1024
_R = 50
_D = 10
_DP = 16
_NPAD = 1024
_LANES = 16


def _tc_body(x_ref, w_ref, b_ref, cpm_ref, huff_ref, y_ref, enc_ref, bidx_ref):
    z = jnp.dot(x_ref[...], w_ref[...], preferred_element_type=jnp.float32)
    y_ref[...] = jax.nn.sigmoid(z + b_ref[...])
    enc_ref[...] = cpm_ref[...] + _NPAD * huff_ref[...]
    bidx_ref[...] = lax.broadcasted_iota(jnp.int32, bidx_ref.shape, 0)


def _tc_stage(x, w_t, b_row, cpm_pad, huff_pad, rows_per_w):
    return pl.pallas_call(
        _tc_body,
        out_shape=(
            jax.ShapeDtypeStruct((_B, _NPAD), jnp.float32),
            jax.ShapeDtypeStruct((_NPAD, _DP), jnp.int32),
            jax.ShapeDtypeStruct((rows_per_w, _R), jnp.int32),
        ),
    )(x, w_t, b_row, cpm_pad, huff_pad)


def _make_sc_gather(n_cores, n_subcores):
    n_workers = n_cores * n_subcores
    rows_per_w = _B // n_workers
    elems_per_w = rows_per_w * _R
    n_chunks = elems_per_w // _LANES
    mesh = plsc.VectorSubcoreMesh(core_axis_name="c", subcore_axis_name="s")

    @functools.partial(
        pl.kernel,
        mesh=mesh,
        out_type=jax.ShapeDtypeStruct((_B * _R,), jnp.float32),
        compiler_params=pltpu.CompilerParams(needs_layout_passes=False),
        scratch_types=[
            pltpu.VMEM((rows_per_w * _NPAD,), jnp.float32),
            pltpu.VMEM((_NPAD * _DP,), jnp.int32),
            pltpu.VMEM((elems_per_w,), jnp.int32),
            pltpu.VMEM((elems_per_w,), jnp.int32),
            pltpu.VMEM((elems_per_w,), jnp.float32),
        ],
    )
    def sc_gather(y_hbm, enc_hbm, tcf_hbm, bidx_hbm, out_hbm,
                  y_v, enc_v, tc_v, bidx_v, out_v):
        wid = lax.axis_index("s") * n_cores + lax.axis_index("c")
        el0 = wid * elems_per_w
        pltpu.sync_copy(y_hbm.at[pl.ds(wid * (rows_per_w * _NPAD),
                                       rows_per_w * _NPAD)], y_v)
        pltpu.sync_copy(enc_hbm, enc_v)
        pltpu.sync_copy(tcf_hbm.at[pl.ds(el0, elems_per_w)], tc_v)
        pltpu.sync_copy(bidx_hbm, bidx_v)

        def chunk(j, carry):
            off = j * _LANES
            tce = tc_v[pl.ds(off, _LANES)] * _DP
            ybase = bidx_v[pl.ds(off, _LANES)] * _NPAD
            prod = None
            for t in range(_D):
                e = plsc.load_gather(enc_v, [tce + t])
                node = jnp.bitwise_and(e, _NPAD - 1)
                c = jnp.right_shift(e, 10).astype(jnp.float32)
                yv = plsc.load_gather(y_v, [ybase + node])
                f = c + yv - 2.0 * c * yv
                prod = f if prod is None else prod * f
            out_v[pl.ds(off, _LANES)] = prod
            return carry

        lax.fori_loop(0, n_chunks, chunk, 0)
        pltpu.sync_copy(out_v, out_hbm.at[pl.ds(el0, elems_per_w)])

    return sc_gather


def kernel(input_vector, target_classes, W, b, class_path_map, huffman_codes):
    n_nodes = W.shape[0]
    w_t = jnp.zeros((input_vector.shape[1], _NPAD), jnp.float32)
    w_t = w_t.at[:, :n_nodes].set(W[:, :, 0].T)
    b_row = jnp.zeros((1, _NPAD), jnp.float32).at[0, :n_nodes].set(b[:, 0])
    cpm_pad = jnp.zeros((_NPAD, _DP), jnp.int32)
    cpm_pad = cpm_pad.at[: class_path_map.shape[0], :_D].set(class_path_map)
    huff_pad = jnp.zeros((_NPAD, _DP), jnp.int32)
    huff_pad = huff_pad.at[: huffman_codes.shape[0], :_D].set(huffman_codes)

    info = plsc.get_sparse_core_info()
    n_workers = info.num_cores * info.num_subcores
    rows_per_w = _B // n_workers

    y_all, enc, bidx = _tc_stage(input_vector, w_t, b_row, cpm_pad, huff_pad,
                                 rows_per_w)
    tcf = target_classes.astype(jnp.int32).reshape(-1)
    out_flat = _make_sc_gather(info.num_cores, info.num_subcores)(
        y_all.reshape(-1), enc.reshape(-1), tcf, bidx.reshape(-1))
    return out_flat.reshape(_B, _R)
